# baseline (speedup 1.0000x reference)
import jax
import jax.numpy as jnp
from jax import lax
from jax.experimental import pallas as pl
from jax.experimental.pallas import tpu as pltpu
from jax.experimental.pallas import tpu_sc as plsc

N_DIMS = 64
SEQ_LEN = 200
BATCH = 4096

_INFO = plsc.get_sparse_core_info()
_NC = _INFO.num_cores
_NS = _INFO.num_subcores
_NW = _NC * _NS

_ROWS = BATCH * SEQ_LEN
_ROWS_PER_W = _ROWS // _NW
_K = 128
_NCHUNK = _ROWS_PER_W // _K


def _body(ex_hbm, cat_hbm, pos_hbm, etab_hbm, ctab_hbm, ptab_hbm, out_hbm,
          ie0, ie1, ic0, ic1, ip0, ip1, bo0, bo1,
          si0, si1, se0, se1, sc0, sc1, sp0, sp1, so0, so1):
    ie = (ie0, ie1)
    ic = (ic0, ic1)
    ip = (ip0, ip1)
    bo = (bo0, bo1)
    si = (si0, si1)
    se = (se0, se1)
    sc = (sc0, sc1)
    sp = (sp0, sp1)
    so = (so0, so1)

    wid = lax.axis_index("s") * _NC + lax.axis_index("c")
    w_base = wid * _ROWS_PER_W

    def issue_idx(ci, b):
        base = w_base + ci * _K
        pltpu.async_copy(ex_hbm.at[pl.ds(base, _K)], ie[b], si[b])
        pltpu.async_copy(cat_hbm.at[pl.ds(base, _K)], ic[b], si[b])
        pltpu.async_copy(pos_hbm.at[pl.ds(base, _K)], ip[b], si[b])

    def wait_idx(ci, b):
        base = w_base + ci * _K
        pltpu.make_async_copy(ex_hbm.at[pl.ds(base, _K)], ie[b], si[b]).wait()
        pltpu.make_async_copy(cat_hbm.at[pl.ds(base, _K)], ic[b], si[b]).wait()
        pltpu.make_async_copy(pos_hbm.at[pl.ds(base, _K)], ip[b], si[b]).wait()

    def wait_e(b):
        pltpu.make_async_copy(etab_hbm.at[ie[b]], bo[b], se[b]).wait()

    def wait_cp(b):
        pltpu.make_async_copy(ctab_hbm.at[ic[b]], bo[b], sc[b]).wait()
        pltpu.make_async_copy(ptab_hbm.at[ip[b]], bo[b], sp[b]).wait()

    def wait_writeback(ci, b):
        base = w_base + ci * _K
        pltpu.make_async_copy(bo[b], out_hbm.at[pl.ds(base, _K)], so[b]).wait()

    issue_idx(0, 0)
    issue_idx(1, 1)
    wait_idx(0, 0)
    pltpu.async_copy(etab_hbm.at[ie[0]], bo[0], se[0])

    def chunk(ci, b):
        nb = 1 - b
        nci = ci + 1
        wait_e(b)
        pltpu.async_copy(ctab_hbm.at[ic[b]], bo[b], sc[b], add=True)
        pltpu.async_copy(ptab_hbm.at[ip[b]], bo[b], sp[b], add=True)

        @pl.when(ci + 2 < _NCHUNK)
        def _():
            issue_idx(ci + 2, b)

        @pl.when(nci < _NCHUNK)
        def _():
            @pl.when(ci >= 1)
            def _():
                wait_writeback(ci - 1, nb)
            wait_idx(nci, nb)
            pltpu.async_copy(etab_hbm.at[ie[nb]], bo[nb], se[nb])

        wait_cp(b)
        base = w_base + ci * _K
        pltpu.async_copy(bo[b], out_hbm.at[pl.ds(base, _K)], so[b])

    def outer(g2, carry):
        for b in range(2):
            chunk(g2 * 2 + b, b)
        return carry

    lax.fori_loop(0, _NCHUNK // 2, outer, 0)

    wait_writeback(_NCHUNK - 2, 0)
    wait_writeback(_NCHUNK - 1, 1)


@jax.jit
def _run(ex_flat, cat_flat, pos_flat, etab, ctab, ptab):
    mesh = plsc.VectorSubcoreMesh(core_axis_name="c", subcore_axis_name="s")
    f = pl.kernel(
        _body,
        out_type=jax.ShapeDtypeStruct((_ROWS, N_DIMS), jnp.float32),
        mesh=mesh,
        scratch_types=[
            pltpu.VMEM((_K,), jnp.int32),
            pltpu.VMEM((_K,), jnp.int32),
            pltpu.VMEM((_K,), jnp.int32),
            pltpu.VMEM((_K,), jnp.int32),
            pltpu.VMEM((_K,), jnp.int32),
            pltpu.VMEM((_K,), jnp.int32),
            pltpu.VMEM((_K, N_DIMS), jnp.float32),
            pltpu.VMEM((_K, N_DIMS), jnp.float32),
            pltpu.SemaphoreType.DMA,
            pltpu.SemaphoreType.DMA,
            pltpu.SemaphoreType.DMA,
            pltpu.SemaphoreType.DMA,
            pltpu.SemaphoreType.DMA,
            pltpu.SemaphoreType.DMA,
            pltpu.SemaphoreType.DMA,
            pltpu.SemaphoreType.DMA,
            pltpu.SemaphoreType.DMA,
            pltpu.SemaphoreType.DMA,
        ],
        compiler_params=pltpu.CompilerParams(use_tc_tiling_on_sc=False),
    )
    return f(ex_flat, cat_flat, pos_flat, etab, ctab, ptab)


def kernel(exercises, categories, exercise_table, category_table, position_table):
    ex_flat = exercises.reshape(-1).astype(jnp.int32)
    cat_flat = categories.reshape(-1).astype(jnp.int32)
    pos_flat = jnp.broadcast_to(
        jnp.arange(SEQ_LEN, dtype=jnp.int32), (BATCH, SEQ_LEN)).reshape(-1)
    out = _run(ex_flat, cat_flat, pos_flat, exercise_table, category_table,
               position_table)
    return out.reshape(BATCH, SEQ_LEN, N_DIMS)


# speedup vs baseline: 1.1931x; 1.1931x over previous
import jax
import jax.numpy as jnp
from jax import lax
from jax.experimental import pallas as pl
from jax.experimental.pallas import tpu as pltpu
from jax.experimental.pallas import tpu_sc as plsc

N_DIMS = 64
SEQ_LEN = 200
BATCH = 4096

_INFO = plsc.get_sparse_core_info()
_NC = _INFO.num_cores
_NS = _INFO.num_subcores
_NW = _NC * _NS

_ROWS = BATCH * SEQ_LEN
_ROWS_PER_W = _ROWS // _NW
_K = 128
_NCHUNK = _ROWS_PER_W // _K
_NBUF = 4
_NWB = 2


def _body(ex_hbm, cat_hbm, etab_hbm, ctab_hbm, ptab_hbm, out_hbm,
          p_v,
          ie0, ie1, ie2, ie3, ic0, ic1, ic2, ic3,
          be0, be1, be2, be3, bc0, bc1, bc2, bc3,
          wo0, wo1,
          si0, si1, si2, si3, sg0, sg1, sg2, sg3, so0, so1):
    ie = (ie0, ie1, ie2, ie3)
    ic = (ic0, ic1, ic2, ic3)
    be = (be0, be1, be2, be3)
    bc = (bc0, bc1, bc2, bc3)
    wo = (wo0, wo1)
    si = (si0, si1, si2, si3)
    sg = (sg0, sg1, sg2, sg3)
    so = (so0, so1)

    wid = lax.axis_index("s") * _NC + lax.axis_index("c")
    w_base = wid * _ROWS_PER_W

    pltpu.sync_copy(ptab_hbm, p_v)

    def issue_idx(ci, b):
        base = w_base + ci * _K
        pltpu.async_copy(ex_hbm.at[pl.ds(base, _K)], ie[b], si[b])
        pltpu.async_copy(cat_hbm.at[pl.ds(base, _K)], ic[b], si[b])

    def wait_idx(ci, b):
        base = w_base + ci * _K
        pltpu.make_async_copy(ex_hbm.at[pl.ds(base, _K)], ie[b], si[b]).wait()
        pltpu.make_async_copy(cat_hbm.at[pl.ds(base, _K)], ic[b], si[b]).wait()

    def issue_gathers(b):
        pltpu.async_copy(etab_hbm.at[ie[b]], be[b], sg[b])
        pltpu.async_copy(ctab_hbm.at[ic[b]], bc[b], sg[b])

    def wait_gathers(b):
        pltpu.make_async_copy(etab_hbm.at[ie[b]], be[b], sg[b]).wait()
        pltpu.make_async_copy(ctab_hbm.at[ic[b]], bc[b], sg[b]).wait()

    def wait_writeback(ci, w):
        base = w_base + ci * _K
        pltpu.make_async_copy(wo[w], out_hbm.at[pl.ds(base, _K)], so[w]).wait()

    for b in range(_NBUF):
        issue_idx(b, b)
    for b in range(_NBUF - 1):
        wait_idx(b, b)
        issue_gathers(b)

    def chunk(ci, b, w):
        wait_gathers(b)

        @pl.when(ci + _NBUF < _NCHUNK)
        def _():
            issue_idx(ci + _NBUF, b)

        fi = ci + _NBUF - 1
        fb = (b + _NBUF - 1) % _NBUF

        @pl.when(fi < _NCHUNK)
        def _():
            wait_idx(fi, fb)
            issue_gathers(fb)

        @pl.when(ci >= _NWB)
        def _():
            wait_writeback(ci - _NWB, w)

        base = w_base + ci * _K

        def row_body(r, carry2):
            s = lax.rem(base + r, SEQ_LEN)
            for d in range(N_DIMS // 16):
                sl = pl.ds(d * 16, 16)
                wo[w][r, sl] = be[b][r, sl] + bc[b][r, sl] + p_v[s, sl]
            return carry2

        lax.fori_loop(0, _K, row_body, 0, unroll=2)
        pltpu.async_copy(wo[w], out_hbm.at[pl.ds(base, _K)], so[w])

    def outer(g, carry):
        for b in range(_NBUF):
            chunk(g * _NBUF + b, b, b % _NWB)
        return carry

    lax.fori_loop(0, _NCHUNK // _NBUF, outer, 0)

    wait_writeback(_NCHUNK - 2, 0)
    wait_writeback(_NCHUNK - 1, 1)


@jax.jit
def _run(ex_flat, cat_flat, etab, ctab, ptab):
    mesh = plsc.VectorSubcoreMesh(core_axis_name="c", subcore_axis_name="s")
    f = pl.kernel(
        _body,
        out_type=jax.ShapeDtypeStruct((_ROWS, N_DIMS), jnp.float32),
        mesh=mesh,
        scratch_types=(
            [pltpu.VMEM((SEQ_LEN, N_DIMS), jnp.float32)]
            + [pltpu.VMEM((_K,), jnp.int32)] * (2 * _NBUF)
            + [pltpu.VMEM((_K, N_DIMS), jnp.float32)] * (2 * _NBUF)
            + [pltpu.VMEM((_K, N_DIMS), jnp.float32)] * _NWB
            + [pltpu.SemaphoreType.DMA] * (_NBUF + _NBUF + _NWB)
        ),
        compiler_params=pltpu.CompilerParams(use_tc_tiling_on_sc=False),
    )
    return f(ex_flat, cat_flat, etab, ctab, ptab)


def kernel(exercises, categories, exercise_table, category_table, position_table):
    ex_flat = exercises.reshape(-1).astype(jnp.int32)
    cat_flat = categories.reshape(-1).astype(jnp.int32)
    out = _run(ex_flat, cat_flat, exercise_table, category_table, position_table)
    return out.reshape(BATCH, SEQ_LEN, N_DIMS)
